# baseline (speedup 1.0000x reference)
import jax
import jax.numpy as jnp
from jax.experimental import pallas as pl
import jax.experimental.pallas.tpu as pltpu

B, H, W = 32, 512, 512
CHUNK = 8
SLOTS = 4
LOOKAHEAD = 2
NSTEPS = B // CHUNK


def _bbox_kernel(in_hbm, seg_hbm, det_ref, bufs, acc, in_sems, out_sems):
    def in_copy(i, slot):
        return pltpu.make_async_copy(
            in_hbm.at[pl.ds(i * CHUNK, CHUNK)],
            bufs.at[slot],
            in_sems.at[slot],
        )

    def out_copy(i, slot):
        return pltpu.make_async_copy(
            bufs.at[slot],
            seg_hbm.at[pl.ds(i * CHUNK, CHUNK), 0],
            out_sems.at[slot],
        )

    for p in range(LOOKAHEAD):
        in_copy(p, p).start()
    acc[...] = jnp.zeros((H, W), jnp.float32)

    def step(i, _):
        s = jax.lax.rem(i, SLOTS)
        in_copy(i, s).wait()
        out_copy(i, s).start()

        @pl.when(i + LOOKAHEAD < NSTEPS)
        def _prefetch():
            nxt = i + LOOKAHEAD
            s2 = jax.lax.rem(nxt, SLOTS)

            @pl.when(nxt >= SLOTS)
            def _reclaim():
                out_copy(nxt - SLOTS, s2).wait()

            in_copy(nxt, s2).start()

        x = bufs[s]
        acc[...] = jnp.maximum(acc[...], jnp.max(jnp.abs(x), axis=0))
        return 0

    jax.lax.fori_loop(0, NSTEPS, step, 0)

    m = acc[...]
    rm = jnp.max(m, axis=1, keepdims=True)
    cm = jnp.max(m, axis=0, keepdims=True)
    hidx = jax.lax.broadcasted_iota(jnp.int32, (H, 1), 0)
    widx = jax.lax.broadcasted_iota(jnp.int32, (1, W), 1)
    has = jnp.max(rm) > 0.0
    ymin = jnp.min(jnp.where(rm > 0.0, hidx, H))
    ymax = jnp.max(jnp.where(rm > 0.0, hidx, -1))
    xmin = jnp.min(jnp.where(cm > 0.0, widx, W))
    xmax = jnp.max(jnp.where(cm > 0.0, widx, -1))
    ymin = jnp.where(has, ymin, 0)
    ymax = jnp.where(has, ymax, 0)
    xmin = jnp.where(has, xmin, 0)
    xmax = jnp.where(has, xmax, 0)
    height = ymax - ymin
    width = xmax - xmin
    cy = ymin + height // 2
    cx = xmin + width // 2
    conf = jnp.clip(100 * height * width, 0, 100)
    lane = jax.lax.broadcasted_iota(jnp.int32, (8, 128), 1)
    det = jnp.where(lane == 0, cx,
          jnp.where(lane == 1, cy,
          jnp.where(lane == 2, width,
          jnp.where(lane == 3, height,
          jnp.where(lane == 4, 5,
          jnp.where(lane == 5, conf, 0))))))
    det_ref[...] = det

    def drain(i, _):
        c = NSTEPS - SLOTS + i
        out_copy(c, jax.lax.rem(c, SLOTS)).wait()
        return 0

    jax.lax.fori_loop(0, SLOTS, drain, 0)


def kernel(inputs):
    seg, det_pad = pl.pallas_call(
        _bbox_kernel,
        in_specs=[pl.BlockSpec(memory_space=pltpu.MemorySpace.HBM)],
        out_specs=[
            pl.BlockSpec(memory_space=pltpu.MemorySpace.HBM),
            pl.BlockSpec(memory_space=pltpu.MemorySpace.VMEM),
        ],
        out_shape=[
            jax.ShapeDtypeStruct((B, 1, H, W), jnp.float32),
            jax.ShapeDtypeStruct((8, 128), jnp.int32),
        ],
        scratch_shapes=[
            pltpu.VMEM((SLOTS, CHUNK, H, W), jnp.float32),
            pltpu.VMEM((H, W), jnp.float32),
            pltpu.SemaphoreType.DMA((SLOTS,)),
            pltpu.SemaphoreType.DMA((SLOTS,)),
        ],
    )(inputs)
    det = jnp.broadcast_to(det_pad[0, :6][None, None, :], (B, 1, 6))
    return det, seg
